# baseline (speedup 1.0000x reference)
import functools

import jax
import jax.numpy as jnp
from jax import lax
from jax.experimental import pallas as pl
from jax.experimental.pallas import tpu as pltpu
from jax.experimental.pallas import tpu_sc as plsc

N_BUCKETS = 1000000
BDIM = 32
DMODEL = 128
MULT = 1056
GUARD = 8


def _gather_body(ids_hbm, table_hbm, emb_hbm, ids_v, idx_v, rows_a, rows_b,
                 gsem_a, gsem_b, stage_sem, *, total, chunk, seq_len, gb, kf):
    nc = 2
    wid = lax.axis_index("s") * nc + lax.axis_index("c")
    rows_per_tile = chunk // seq_len
    base = wid * chunk
    bigb = gb * kf
    nbig = chunk // bigb

    pltpu.async_copy(
        ids_hbm.at[pl.ds(wid * rows_per_tile, rows_per_tile), :],
        ids_v.at[pl.ds(0, rows_per_tile), pl.ds(GUARD, seq_len)],
        stage_sem,
    ).wait()

    lanes = lax.iota(jnp.int32, 16)
    tstarts = list(range(0, seq_len - 16, 16)) + [seq_len - 16]

    def hash_body(r, carry):
        for t0 in tstarts:
            curr = ids_v[r, pl.ds(t0 + GUARD, 16)]
            prev = ids_v[r, pl.ds(t0 + GUARD - 1, 16)]
            if t0 == 0:
                prev = jnp.where(lanes == 0, 0, prev)
            h = (prev * MULT + curr) % N_BUCKETS
            idx_v[pl.ds(r * seq_len + t0, 16)] = h
        return carry

    lax.fori_loop(0, rows_per_tile, hash_body, 0)

    def fire(buf, gsem, c):
        for k in range(kf):
            idx_slice = idx_v.at[pl.ds(c * bigb + k * gb, gb)]
            pltpu.async_copy(
                table_hbm.at[idx_slice], buf.at[pl.ds(k * gb, gb)], gsem
            )

    def drain(buf, gsem):
        pltpu.make_async_copy(emb_hbm.at[pl.ds(0, bigb)], buf, gsem).wait()

    def out_copy(buf, c):
        pltpu.sync_copy(buf, emb_hbm.at[pl.ds(base + c * bigb, bigb)])

    fire(rows_a, gsem_a, 0)

    def pipe_body(p, carry):
        ca = 2 * p
        fire(rows_b, gsem_b, ca + 1)
        drain(rows_a, gsem_a)
        out_copy(rows_a, ca)
        fire(rows_a, gsem_a, ca + 2)
        drain(rows_b, gsem_b)
        out_copy(rows_b, ca + 1)
        return carry

    lax.fori_loop(0, (nbig - 1) // 2, pipe_body, 0)

    drain(rows_a, gsem_a)
    out_copy(rows_a, nbig - 1)


def _proj_body(x_ref, w_ref, o_ref, *, blk):
    y = lax.dot_general(
        x_ref[...], w_ref[...],
        (((1,), (0,)), ((), ())),
        preferred_element_type=jnp.float32,
    )
    o_ref[...] = y.reshape(4 * blk, DMODEL)


@jax.jit
def kernel(input_ids, bigram_table, proj_w):
    batch, seq_len = input_ids.shape
    total = batch * seq_len
    total4 = total // 4
    nw = 32
    chunk = total // nw
    gb = 128
    kf = 8

    mesh = plsc.VectorSubcoreMesh(core_axis_name="c", subcore_axis_name="s")
    sc_gather = functools.partial(
        pl.kernel,
        mesh=mesh,
        out_type=jax.ShapeDtypeStruct((total, BDIM), jnp.float32),
        scratch_types=[
            pltpu.VMEM((batch // nw, seq_len + GUARD), jnp.int32),
            pltpu.VMEM((chunk,), jnp.int32),
            pltpu.VMEM((gb * kf, BDIM), jnp.float32),
            pltpu.VMEM((gb * kf, BDIM), jnp.float32),
            pltpu.SemaphoreType.DMA,
            pltpu.SemaphoreType.DMA,
            pltpu.SemaphoreType.DMA,
        ],
        compiler_params=pltpu.CompilerParams(use_tc_tiling_on_sc=False),
    )(functools.partial(_gather_body, total=total, chunk=chunk,
                        seq_len=seq_len, gb=gb, kf=kf))

    emb4 = sc_gather(input_ids, bigram_table).reshape(total4, DMODEL)

    wt = proj_w.T
    w4 = jnp.zeros((DMODEL, 4 * DMODEL), jnp.float32)
    for k in range(4):
        w4 = lax.dynamic_update_slice(w4, wt, (BDIM * k, DMODEL * k))

    blk = 512
    out = pl.pallas_call(
        functools.partial(_proj_body, blk=blk),
        grid=(total4 // blk,),
        in_specs=[
            pl.BlockSpec((blk, DMODEL), lambda i: (i, 0)),
            pl.BlockSpec((DMODEL, 4 * DMODEL), lambda i: (0, 0)),
        ],
        out_specs=pl.BlockSpec((4 * blk, DMODEL), lambda i: (i, 0)),
        out_shape=jax.ShapeDtypeStruct((total, DMODEL), jnp.float32),
    )(emb4, w4)

    return out.reshape(batch, seq_len, DMODEL)


# speedup vs baseline: 1.0515x; 1.0515x over previous
import functools

import jax
import jax.numpy as jnp
from jax import lax
from jax.experimental import pallas as pl
from jax.experimental.pallas import tpu as pltpu
from jax.experimental.pallas import tpu_sc as plsc

N_BUCKETS = 1000000
BDIM = 32
DMODEL = 128
MULT = 1056
GUARD = 8


def _gather_body(ids_hbm, table_hbm, emb_hbm, ids_v, idx_v, rows_a, rows_b,
                 gsem_a, gsem_b, stage_sem, *, total, chunk, seq_len, gb, kf):
    nc = 2
    wid = lax.axis_index("s") * nc + lax.axis_index("c")
    rows_per_tile = chunk // seq_len
    base = wid * chunk
    bigb = gb * kf
    nbig = chunk // bigb

    pltpu.async_copy(
        ids_hbm.at[pl.ds(wid * rows_per_tile, rows_per_tile), :],
        ids_v.at[pl.ds(0, rows_per_tile), pl.ds(GUARD, seq_len)],
        stage_sem,
    ).wait()

    lanes = lax.iota(jnp.int32, 16)
    tstarts = list(range(0, seq_len - 16, 16)) + [seq_len - 16]

    def hash_body(r, carry):
        for t0 in tstarts:
            curr = ids_v[r, pl.ds(t0 + GUARD, 16)]
            prev = ids_v[r, pl.ds(t0 + GUARD - 1, 16)]
            if t0 == 0:
                prev = jnp.where(lanes == 0, 0, prev)
            h = (prev * MULT + curr) % N_BUCKETS
            idx_v[pl.ds(r * seq_len + t0, 16)] = h
        return carry

    lax.fori_loop(0, rows_per_tile, hash_body, 0)

    def fire(buf, gsem, c):
        for k in range(kf):
            idx_slice = idx_v.at[pl.ds(c * bigb + k * gb, gb)]
            pltpu.async_copy(
                table_hbm.at[idx_slice], buf.at[pl.ds(k * gb, gb)], gsem
            )

    def drain(buf, gsem):
        pltpu.make_async_copy(emb_hbm.at[pl.ds(0, bigb)], buf, gsem).wait()

    def out_copy(buf, c):
        pltpu.sync_copy(buf, emb_hbm.at[pl.ds(base + c * bigb, bigb)])

    fire(rows_a, gsem_a, 0)

    def pipe_body(p, carry):
        ca = 2 * p
        fire(rows_b, gsem_b, ca + 1)
        drain(rows_a, gsem_a)
        out_copy(rows_a, ca)
        fire(rows_a, gsem_a, ca + 2)
        drain(rows_b, gsem_b)
        out_copy(rows_b, ca + 1)
        return carry

    lax.fori_loop(0, (nbig - 1) // 2, pipe_body, 0)

    drain(rows_a, gsem_a)
    out_copy(rows_a, nbig - 1)


def _proj_body(x_ref, w_ref, o_ref, *, blk):
    y = lax.dot_general(
        x_ref[...], w_ref[...],
        (((1,), (0,)), ((), ())),
        preferred_element_type=jnp.float32,
    )
    o_ref[...] = y.reshape(4 * blk, DMODEL)


def _proj_body2(prev_hbm, x_ref, w_ref, o_ref, *, blk):
    del prev_hbm
    y = lax.dot_general(
        x_ref[...], w_ref[...],
        (((1,), (0,)), ((), ())),
        preferred_element_type=jnp.float32,
    )
    o_ref[...] = y.reshape(4 * blk, DMODEL)


@jax.jit
def kernel(input_ids, bigram_table, proj_w):
    batch, seq_len = input_ids.shape
    total = batch * seq_len
    total4 = total // 4
    nw = 32
    gb = 128
    kf = 4

    hbatch = batch // 2
    htotal = total // 2
    chunk = htotal // nw

    mesh = plsc.VectorSubcoreMesh(core_axis_name="c", subcore_axis_name="s")
    sc_gather = functools.partial(
        pl.kernel,
        mesh=mesh,
        out_type=jax.ShapeDtypeStruct((htotal, BDIM), jnp.float32),
        scratch_types=[
            pltpu.VMEM((hbatch // nw, seq_len + GUARD), jnp.int32),
            pltpu.VMEM((chunk,), jnp.int32),
            pltpu.VMEM((gb * kf, BDIM), jnp.float32),
            pltpu.VMEM((gb * kf, BDIM), jnp.float32),
            pltpu.SemaphoreType.DMA,
            pltpu.SemaphoreType.DMA,
            pltpu.SemaphoreType.DMA,
        ],
        compiler_params=pltpu.CompilerParams(use_tc_tiling_on_sc=False),
    )(functools.partial(_gather_body, total=htotal, chunk=chunk,
                        seq_len=seq_len, gb=gb, kf=kf))

    emb4_a = sc_gather(input_ids[:hbatch], bigram_table).reshape(
        htotal // 4, DMODEL)
    emb4_b = sc_gather(input_ids[hbatch:], bigram_table).reshape(
        htotal // 4, DMODEL)

    wt = proj_w.T
    w4 = jnp.zeros((DMODEL, 4 * DMODEL), jnp.float32)
    for k in range(4):
        w4 = lax.dynamic_update_slice(w4, wt, (BDIM * k, DMODEL * k))

    blk = 512
    hgrid = htotal // 4 // blk
    out1 = pl.pallas_call(
        functools.partial(_proj_body, blk=blk),
        grid=(hgrid,),
        in_specs=[
            pl.BlockSpec((blk, DMODEL), lambda i: (i, 0)),
            pl.BlockSpec((DMODEL, 4 * DMODEL), lambda i: (0, 0)),
        ],
        out_specs=pl.BlockSpec((4 * blk, DMODEL), lambda i: (i, 0)),
        out_shape=jax.ShapeDtypeStruct((total, DMODEL), jnp.float32),
    )(emb4_a, w4)

    out = pl.pallas_call(
        functools.partial(_proj_body2, blk=blk),
        grid=(hgrid,),
        in_specs=[
            pl.BlockSpec(memory_space=pl.ANY),
            pl.BlockSpec((blk, DMODEL), lambda i: (i, 0)),
            pl.BlockSpec((DMODEL, 4 * DMODEL), lambda i: (0, 0)),
        ],
        out_specs=pl.BlockSpec((4 * blk, DMODEL),
                               lambda i: (i + hgrid, 0)),
        out_shape=jax.ShapeDtypeStruct((total, DMODEL), jnp.float32),
        input_output_aliases={0: 0},
    )(out1, emb4_b, w4)

    return out.reshape(batch, seq_len, DMODEL)
